# Initial kernel scaffold; baseline (speedup 1.0000x reference)
import jax
import jax.numpy as jnp
from jax.experimental import pallas as pl


def kernel(x_building, building_to_cable, cable_to_transformer, W1, b1, W2, b2, W3, b3, W4, b4):
    raise NotImplementedError("write your pallas kernel here")


# speedup vs baseline: 3.9353x; 3.9353x over previous
import functools

import jax
import jax.numpy as jnp
from jax import lax
from jax.experimental import pallas as pl
from jax.experimental.pallas import tpu as pltpu
from jax.experimental.pallas import tpu_sc as plsc

N_BUILDING = 100000
N_CABLE = 10000
N_TRANSFORMER = 1000
D = 128

NCORES = 2
NSUB = 16
NW = NCORES * NSUB

SUB = 80
NSUBCH = 5
CH1 = SUB * NSUBCH
NCHUNK1 = N_BUILDING // CH1
ITER1 = -(-NCHUNK1 // NSUB)
HALF_SEG = N_CABLE // NCORES
ACC1 = HALF_SEG + 8
STRIPE1I = 320
LAST1I = ACC1 - (NSUB - 1) * STRIPE1I
STRIPE1O = 312
LAST1O = HALF_SEG - (NSUB - 1) * STRIPE1O

NCHUNK2 = N_CABLE // SUB
ITER2 = -(-NCHUNK2 // NW)
STRIPE2 = 64
LAST2 = N_TRANSFORMER - (NSUB - 1) * STRIPE2


def _mlp1_body(x_ref, w_ref, b_ref, o_ref):
    h = jnp.dot(x_ref[...], w_ref[...], preferred_element_type=jnp.float32)
    o_ref[...] = jnp.maximum(h + b_ref[...], 0.0)


def _tc_relu_linear(x, w_t, b, block_rows):
    n = x.shape[0]
    return pl.pallas_call(
        _mlp1_body,
        grid=(n // block_rows,),
        in_specs=[
            pl.BlockSpec((block_rows, D), lambda i: (i, 0)),
            pl.BlockSpec((D, D), lambda i: (0, 0)),
            pl.BlockSpec((1, D), lambda i: (0, 0)),
        ],
        out_specs=pl.BlockSpec((block_rows, D), lambda i: (i, 0)),
        out_shape=jax.ShapeDtypeStruct((n, D), jnp.float32),
    )(x, w_t, b)


def _combine_body(s_ref, c_ref, w2t_ref, b2_ref, w3t_ref, b3_ref,
                  xc_ref, g_ref):
    s = s_ref[...]
    cnt = jnp.sum(c_ref[...], axis=1)[:, None]
    mean = s / jnp.maximum(cnt, 1.0)
    xc = jnp.dot(mean, w2t_ref[...], preferred_element_type=jnp.float32)
    xc = jnp.where(cnt > 0.0, xc + b2_ref[...], 0.0)
    xc_ref[...] = xc
    g = jnp.dot(xc, w3t_ref[...], preferred_element_type=jnp.float32)
    g_ref[...] = jnp.maximum(g + b3_ref[...], 0.0)


def _final_body(s_ref, c_ref, w4t_ref, b4_ref, o_ref):
    s = s_ref[0] + s_ref[1]
    cnt = jnp.sum(c_ref[...], axis=0)[:, None]
    mean = s / jnp.maximum(cnt, 1.0)
    o = jnp.dot(mean, w4t_ref[...], preferred_element_type=jnp.float32)
    o_ref[...] = jnp.where(cnt > 0.0, o + b4_ref[...], 0.0)


_MESH = plsc.VectorSubcoreMesh(core_axis_name="c", subcore_axis_name="s")


@functools.partial(
    pl.kernel,
    mesh=_MESH,
    out_type=(
        jax.ShapeDtypeStruct((N_CABLE, D), jnp.float32),
        jax.ShapeDtypeStruct((NW, ACC1), jnp.float32),
    ),
    scratch_types=[
        pltpu.VMEM_SHARED((ACC1, D), jnp.float32),
        pltpu.VMEM((CH1, D), jnp.float32),
        pltpu.VMEM((NSUBCH, SUB), jnp.int32),
        pltpu.VMEM((NSUBCH, SUB), jnp.int32),
        pltpu.VMEM((ACC1,), jnp.float32),
    ],
    compiler_params=pltpu.CompilerParams(needs_layout_passes=False),
)
def _segsum1(rows_hbm, idx_hbm, zrow_hbm, zcnt_hbm,
             sums_out, cnt_out, acc, rows_v, idx_v, sidx_v, cnt_v):
    c = lax.axis_index("c")
    s = lax.axis_index("s")
    lo = c * HALF_SEG
    hi = lo + HALF_SEG

    zb = s * STRIPE1I

    @pl.when(s < NSUB - 1)
    def _():
        pltpu.sync_copy(zrow_hbm.at[pl.ds(zb, STRIPE1I)],
                        acc.at[pl.ds(zb, STRIPE1I)])

    @pl.when(s == NSUB - 1)
    def _():
        pltpu.sync_copy(zrow_hbm.at[pl.ds(zb, LAST1I)],
                        acc.at[pl.ds(zb, LAST1I)])

    pltpu.sync_copy(zcnt_hbm, cnt_v)
    plsc.subcore_barrier()

    ones16 = jnp.ones((16,), jnp.float32)
    for k in range(ITER1):
        chunk = s + NSUB * k

        @pl.when(chunk < NCHUNK1)
        def _(chunk=chunk):
            pltpu.sync_copy(idx_hbm.at[chunk], idx_v)
            first = jnp.min(idx_v[0, pl.ds(0, 16)])
            last = jnp.max(idx_v[NSUBCH - 1, pl.ds(SUB - 16, 16)])

            @pl.when((first < hi) & (last >= lo))
            def _():
                pltpu.sync_copy(rows_hbm.at[pl.ds(chunk * CH1, CH1)], rows_v)
                for j in range(NSUBCH):
                    for t in range(SUB // 16):
                        v = idx_v[j, pl.ds(t * 16, 16)]
                        inr = (v >= lo) & (v < hi)
                        vloc = jnp.where(inr, v - lo, HALF_SEG)
                        sidx_v[j, pl.ds(t * 16, 16)] = vloc
                        plsc.addupdate_scatter(cnt_v, [vloc], ones16)
                for j in range(NSUBCH):
                    pltpu.sync_copy(rows_v.at[pl.ds(j * SUB, SUB)],
                                    acc.at[sidx_v.at[j]], add=True)

    plsc.subcore_barrier()

    wb = s * STRIPE1O
    ob = c * HALF_SEG + wb

    @pl.when(s < NSUB - 1)
    def _():
        pltpu.sync_copy(acc.at[pl.ds(wb, STRIPE1O)],
                        sums_out.at[pl.ds(ob, STRIPE1O)])

    @pl.when(s == NSUB - 1)
    def _():
        pltpu.sync_copy(acc.at[pl.ds(wb, LAST1O)],
                        sums_out.at[pl.ds(ob, LAST1O)])

    pltpu.sync_copy(cnt_v, cnt_out.at[c * NSUB + s])


@functools.partial(
    pl.kernel,
    mesh=_MESH,
    out_type=(
        jax.ShapeDtypeStruct((NCORES * N_TRANSFORMER, D), jnp.float32),
        jax.ShapeDtypeStruct((NW, N_TRANSFORMER), jnp.float32),
    ),
    scratch_types=[
        pltpu.VMEM_SHARED((N_TRANSFORMER, D), jnp.float32),
        pltpu.VMEM((SUB, D), jnp.float32),
        pltpu.VMEM((SUB,), jnp.int32),
        pltpu.VMEM((N_TRANSFORMER,), jnp.float32),
    ],
    compiler_params=pltpu.CompilerParams(needs_layout_passes=False),
)
def _segsum2(rows_hbm, idx_hbm, zrow_hbm, zcnt_hbm,
             sums_out, cnt_out, acc, rows_v, idx_v, cnt_v):
    c = lax.axis_index("c")
    s = lax.axis_index("s")
    wid = s * NCORES + c

    zb = s * STRIPE2

    @pl.when(s < NSUB - 1)
    def _():
        pltpu.sync_copy(zrow_hbm.at[pl.ds(zb, STRIPE2)],
                        acc.at[pl.ds(zb, STRIPE2)])

    @pl.when(s == NSUB - 1)
    def _():
        pltpu.sync_copy(zrow_hbm.at[pl.ds(zb, LAST2)],
                        acc.at[pl.ds(zb, LAST2)])

    pltpu.sync_copy(zcnt_hbm, cnt_v)
    plsc.subcore_barrier()

    ones16 = jnp.ones((16,), jnp.float32)
    for k in range(ITER2):
        chunk = wid + NW * k

        @pl.when(chunk < NCHUNK2)
        def _(chunk=chunk):
            row0 = chunk * SUB
            pltpu.sync_copy(rows_hbm.at[pl.ds(row0, SUB)], rows_v)
            pltpu.sync_copy(idx_hbm.at[pl.ds(row0, SUB)], idx_v)
            for t in range(SUB // 16):
                v = idx_v[pl.ds(t * 16, 16)]
                plsc.addupdate_scatter(cnt_v, [v], ones16)
            pltpu.sync_copy(rows_v, acc.at[idx_v], add=True)

    plsc.subcore_barrier()

    ob = c * N_TRANSFORMER + zb

    @pl.when(s < NSUB - 1)
    def _():
        pltpu.sync_copy(acc.at[pl.ds(zb, STRIPE2)],
                        sums_out.at[pl.ds(ob, STRIPE2)])

    @pl.when(s == NSUB - 1)
    def _():
        pltpu.sync_copy(acc.at[pl.ds(zb, LAST2)],
                        sums_out.at[pl.ds(ob, LAST2)])

    pltpu.sync_copy(cnt_v, cnt_out.at[c * NSUB + s])


def kernel(x_building, building_to_cable, cable_to_transformer,
           W1, b1, W2, b2, W3, b3, W4, b4):
    idx_b = building_to_cable.astype(jnp.int32).reshape(NCHUNK1, NSUBCH, SUB)
    idx_c = cable_to_transformer.astype(jnp.int32)
    zrow = jnp.zeros((ACC1, D), jnp.float32)
    zcnt1 = jnp.zeros((ACC1,), jnp.float32)
    zcnt2 = jnp.zeros((N_TRANSFORMER,), jnp.float32)

    a = _tc_relu_linear(x_building, W1.T, b1.reshape(1, D), 2000)

    sums_b, cnt32_b = _segsum1(a, idx_b, zrow, zcnt1)
    cnt_b = jnp.concatenate([cnt32_b[:NSUB, :HALF_SEG],
                             cnt32_b[NSUB:, :HALF_SEG]], axis=1).T

    blk = 1000
    x_cable, g = pl.pallas_call(
        _combine_body,
        grid=(N_CABLE // blk,),
        in_specs=[
            pl.BlockSpec((blk, D), lambda i: (i, 0)),
            pl.BlockSpec((blk, NSUB), lambda i: (i, 0)),
            pl.BlockSpec((D, D), lambda i: (0, 0)),
            pl.BlockSpec((1, D), lambda i: (0, 0)),
            pl.BlockSpec((D, D), lambda i: (0, 0)),
            pl.BlockSpec((1, D), lambda i: (0, 0)),
        ],
        out_specs=[
            pl.BlockSpec((blk, D), lambda i: (i, 0)),
            pl.BlockSpec((blk, D), lambda i: (i, 0)),
        ],
        out_shape=[
            jax.ShapeDtypeStruct((N_CABLE, D), jnp.float32),
            jax.ShapeDtypeStruct((N_CABLE, D), jnp.float32),
        ],
    )(sums_b, cnt_b, W2.T, b2.reshape(1, D), W3.T, b3.reshape(1, D))

    sums_t, cnt32_t = _segsum2(g, idx_c, zrow[:N_TRANSFORMER], zcnt2)
    sums_t = sums_t.reshape(NCORES, N_TRANSFORMER, D)

    x_transformer = pl.pallas_call(
        _final_body,
        grid=(1,),
        in_specs=[
            pl.BlockSpec((NCORES, N_TRANSFORMER, D), lambda i: (0, 0, 0)),
            pl.BlockSpec((NW, N_TRANSFORMER), lambda i: (0, 0)),
            pl.BlockSpec((D, D), lambda i: (0, 0)),
            pl.BlockSpec((1, D), lambda i: (0, 0)),
        ],
        out_specs=pl.BlockSpec((N_TRANSFORMER, D), lambda i: (0, 0)),
        out_shape=jax.ShapeDtypeStruct((N_TRANSFORMER, D), jnp.float32),
    )(sums_t, cnt32_t, W4.T, b4.reshape(1, D))

    return (x_cable, x_transformer)
